# baseline (speedup 1.0000x reference)
import functools

import jax
import jax.numpy as jnp
import numpy as np
from jax import lax
from jax.experimental import pallas as pl
from jax.experimental.pallas import tpu as pltpu
from jax.experimental.pallas import tpu_sc as plsc

M = 512
K = 32
NUM_FREQ = 10
N = 20000
NPAD = 20480
ROWS = 160

_INTERPRET = False


def _fps_body(init_ref, px_ref, py_ref, cent_ref, dist_ref):
    px = px_ref[:]
    py = py_ref[:]
    row = lax.broadcasted_iota(jnp.int32, (ROWS, 128), 0)
    col = lax.broadcasted_iota(jnp.int32, (ROWS, 128), 1)
    fi = row * 128 + col
    valid = fi < N

    init = init_ref[0]
    m0 = fi == init
    cx0 = jnp.sum(jnp.where(m0, px, 0.0))
    cy0 = jnp.sum(jnp.where(m0, py, 0.0))
    cent_ref[0, 0] = cx0
    cent_ref[0, 1] = cy0
    dist_ref[:] = jnp.where(valid, jnp.inf, 0.0)

    coli1 = lax.broadcasted_iota(jnp.int32, (1, 128), 1)

    def body(i, carry):
        cx, cy = carry
        dx = px - cx
        dy = py - cy
        nd2 = dx * dx + dy * dy + jnp.float32(1e-12)
        d2 = jnp.minimum(dist_ref[:], nd2)
        dist_ref[:] = d2
        s = jnp.sqrt(d2)
        mx = jnp.max(s)
        rmin = jnp.min(jnp.where(s == mx, row, ROWS), axis=0, keepdims=True)
        r0 = jnp.min(rmin)
        c0 = jnp.min(jnp.where(rmin == r0, coli1, 128))
        colmask = coli1 == c0
        pxr = px_ref[pl.ds(r0, 1), :]
        pyr = py_ref[pl.ds(r0, 1), :]
        ncx = jnp.sum(jnp.where(colmask, pxr, 0.0))
        ncy = jnp.sum(jnp.where(colmask, pyr, 0.0))
        drow = dist_ref[pl.ds(r0, 1), :]
        dist_ref[pl.ds(r0, 1), :] = jnp.where(colmask, 0.0, drow)
        cent_ref[i, 0] = ncx
        cent_ref[i, 1] = ncy
        return (ncx, ncy)

    lax.fori_loop(1, M, body, (cx0, cy0))


def _fps(px, py, init):
    return pl.pallas_call(
        _fps_body,
        out_shape=jax.ShapeDtypeStruct((M, 2), jnp.float32),
        in_specs=[
            pl.BlockSpec(memory_space=pltpu.SMEM),
            pl.BlockSpec(memory_space=pltpu.VMEM),
            pl.BlockSpec(memory_space=pltpu.VMEM),
        ],
        out_specs=pl.BlockSpec(memory_space=pltpu.SMEM),
        scratch_shapes=[pltpu.VMEM((ROWS, 128), jnp.float32)],
        interpret=_INTERPRET,
    )(init, px, py)


_NB = 256


def _nn_body(px_ref, py_ref, cx_ref, cy_ref, out_ref):
    dx = px_ref[:] - cx_ref[:]
    dy = py_ref[:] - cy_ref[:]
    d2 = dx * dx + dy * dy
    m = jnp.min(d2, axis=1, keepdims=True)
    lane = lax.broadcasted_iota(jnp.int32, (_NB, M), 1)
    out_ref[:] = jnp.min(jnp.where(d2 == m, lane, M), axis=1, keepdims=True)


def _nearest(px_col, py_col, cx_row, cy_row):
    grid = NPAD // _NB
    return pl.pallas_call(
        _nn_body,
        grid=(grid,),
        out_shape=jax.ShapeDtypeStruct((NPAD, 1), jnp.int32),
        in_specs=[
            pl.BlockSpec((_NB, 1), lambda i: (i, 0)),
            pl.BlockSpec((_NB, 1), lambda i: (i, 0)),
            pl.BlockSpec((1, M), lambda i: (0, 0)),
            pl.BlockSpec((1, M), lambda i: (0, 0)),
        ],
        out_specs=pl.BlockSpec((_NB, 1), lambda i: (i, 0)),
        interpret=_INTERPRET,
    )(px_col, py_col, cx_row, cy_row)


_NW = 16
_CH = NPAD // _NW
_CPW = M // _NW
_SELPAD = M * K + 256


def _sc_group_body(near_hbm, ptx_hbm, pty_hbm, cx_hbm, cy_hbm,
                   relx_hbm, rely_hbm,
                   near_v, hist_v, allhist_v, start_v, cnts_v,
                   slot_v, val_v, selblk_v, ptx_v, pty_v,
                   cxl_v, cyl_v, relx_v, rely_v,
                   hist_sh, sel_sh, sem1, sem2):
    wid = lax.axis_index("s")
    base = wid * _CH
    cp1 = pltpu.async_copy(ptx_hbm, ptx_v, sem1)
    cp2 = pltpu.async_copy(pty_hbm, pty_v, sem2)
    pltpu.sync_copy(near_hbm.at[pl.ds(base, _CH)], near_v)
    pltpu.sync_copy(cx_hbm.at[pl.ds(wid * _CPW, _CPW)], cxl_v)
    pltpu.sync_copy(cy_hbm.at[pl.ds(wid * _CPW, _CPW)], cyl_v)

    zero16 = jnp.zeros((16,), jnp.int32)
    lane = lax.broadcasted_iota(jnp.int32, (16,), 0)
    mask0 = lane == 0
    for j in range(M // 16):
        hist_v[pl.ds(j * 16, 16)] = zero16

    def hbody(i, carry):
        c = plsc.load_gather(near_v, [zero16 + i])[0]
        valid = (c < M).astype(jnp.int32)
        cm16 = zero16 + jnp.minimum(c, M - 1)
        h = plsc.load_gather(hist_v, [cm16])[0]
        plsc.store_scatter(hist_v, [cm16], zero16 + h + valid, mask=mask0)
        return carry

    lax.fori_loop(0, _CH, hbody, 0)

    pltpu.sync_copy(hist_v, hist_sh.at[wid])
    plsc.subcore_barrier()
    pltpu.sync_copy(hist_sh, allhist_v)

    for j in range(M // 16):
        start_v[pl.ds(j * 16, 16)] = zero16
        cnts_v[pl.ds(j * 16, 16)] = zero16

    def pbody(t, carry):
        mlt = (t < wid).astype(jnp.int32)
        for j in range(M // 16):
            sl = pl.ds(j * 16, 16)
            v = allhist_v[t, sl]
            cnts_v[sl] = cnts_v[sl] + v
            start_v[sl] = start_v[sl] + v * mlt
        return carry

    lax.fori_loop(0, _NW, pbody, 0)

    dummy = M * K + 8 * wid
    dummy16 = zero16 + dummy
    for j in range(_CH // 128):
        for h in range(8):
            slot_v[j, pl.ds(h * 16, 16)] = dummy16
            val_v[j, pl.ds(h * 16, 16)] = zero16

    def bbody(i, w):
        c = plsc.load_gather(near_v, [zero16 + i])[0]
        valid = (c < M).astype(jnp.int32)
        cm = jnp.minimum(c, M - 1)
        cm16 = zero16 + cm
        r = plsc.load_gather(start_v, [cm16])[0]
        plsc.store_scatter(start_v, [cm16], zero16 + r + valid, mask=mask0)
        take = jnp.logical_and(valid == 1, r < K)
        takei = take.astype(jnp.int32)
        slot = jnp.where(take, cm * K + jnp.minimum(r, K - 1), dummy)
        val = jnp.where(take, base + i, 0)
        wr = zero16 + (w >> 7)
        wc = zero16 + (w & 127)
        plsc.store_scatter(slot_v, [wr, wc], zero16 + slot, mask=mask0)
        plsc.store_scatter(val_v, [wr, wc], zero16 + val, mask=mask0)
        return w + takei

    lax.fori_loop(0, _CH, bbody, 0)

    for j in range(_CH // 128):
        pltpu.sync_copy(val_v.at[j], sel_sh.at[slot_v.at[j]])
    plsc.subcore_barrier()

    pltpu.sync_copy(sel_sh.at[pl.ds(wid * _CPW * K, _CPW * K)], selblk_v)
    cp1.wait()
    cp2.wait()

    cxl0 = cxl_v[pl.ds(0, 16)]
    cxl1 = cxl_v[pl.ds(16, 16)]
    cyl0 = cyl_v[pl.ds(0, 16)]
    cyl1 = cyl_v[pl.ds(16, 16)]
    for cl in range(_CPW):
        cnt = plsc.load_gather(cnts_v, [zero16 + wid * _CPW + cl])[0]
        cntc = jnp.minimum(cnt, K)
        last = plsc.load_gather(
            selblk_v, [zero16 + cl * K + jnp.maximum(cntc - 1, 0)])[0]
        cx = cxl0[cl] if cl < 16 else cxl1[cl - 16]
        cy = cyl0[cl] if cl < 16 else cyl1[cl - 16]
        emptyc = cnt == 0
        for j in range(K // 16):
            off = pl.ds(cl * K + j * 16, 16)
            kvec = lane + j * 16
            s = jnp.where(kvec < cntc, selblk_v[off], last)
            s = jnp.clip(s, 0, N - 1)
            gx = plsc.load_gather(ptx_v, [s])
            gy = plsc.load_gather(pty_v, [s])
            first = jnp.logical_and(emptyc, kvec == 0)
            rx = jnp.where(first, 0.0, jnp.where(emptyc, 0.0 - cx, gx - cx))
            ry = jnp.where(first, 0.0, jnp.where(emptyc, 0.0 - cy, gy - cy))
            relx_v[off] = rx
            rely_v[off] = ry

    pltpu.sync_copy(relx_v, relx_hbm.at[pl.ds(wid * _CPW * K, _CPW * K)])
    pltpu.sync_copy(rely_v, rely_hbm.at[pl.ds(wid * _CPW * K, _CPW * K)])


def _sc_group(near_pad, ptx_pad, pty_pad, centx, centy):
    mesh = plsc.VectorSubcoreMesh(core_axis_name="c", subcore_axis_name="s",
                                  num_cores=1)
    return pl.kernel(
        _sc_group_body,
        out_type=[jax.ShapeDtypeStruct((M * K,), jnp.float32),
                  jax.ShapeDtypeStruct((M * K,), jnp.float32)],
        mesh=mesh,
        compiler_params=pltpu.CompilerParams(needs_layout_passes=False),
        scratch_types=[
            pltpu.VMEM((_CH,), jnp.int32),
            pltpu.VMEM((M,), jnp.int32),
            pltpu.VMEM((_NW, M), jnp.int32),
            pltpu.VMEM((M,), jnp.int32),
            pltpu.VMEM((M,), jnp.int32),
            pltpu.VMEM((_CH // 128, 128), jnp.int32),
            pltpu.VMEM((_CH // 128, 128), jnp.int32),
            pltpu.VMEM((_CPW * K,), jnp.int32),
            pltpu.VMEM((NPAD,), jnp.float32),
            pltpu.VMEM((NPAD,), jnp.float32),
            pltpu.VMEM((_CPW,), jnp.float32),
            pltpu.VMEM((_CPW,), jnp.float32),
            pltpu.VMEM((_CPW * K,), jnp.float32),
            pltpu.VMEM((_CPW * K,), jnp.float32),
            pltpu.VMEM_SHARED((_NW, M), jnp.int32),
            pltpu.VMEM_SHARED((_SELPAD,), jnp.int32),
            pltpu.SemaphoreType.DMA,
            pltpu.SemaphoreType.DMA,
        ],
    )(near_pad, ptx_pad, pty_pad, centx, centy)


def _mlp_body(relx_ref, rely_ref, w1_ref, b1_ref, g1_ref, be1_ref,
              w2_ref, b2_ref, g2_ref, be2_ref, w3_ref, b3_ref, out_ref):
    rx = relx_ref[:]
    ry = rely_ref[:]
    ii = lax.broadcasted_iota(jnp.int32, (1, 2 * NUM_FREQ), 1)
    freq = lax.shift_left(1, ii // 2).astype(jnp.float32) * jnp.float32(np.pi)
    rsel = jnp.where((ii % 2) == 0, rx, ry)
    args = rsel * freq
    x = jnp.concatenate([jnp.sin(args), jnp.cos(args)], axis=1)

    def dense_bn_relu(h, w_ref, b_ref, g_ref, be_ref):
        h = jnp.dot(h, w_ref[:], preferred_element_type=jnp.float32) + b_ref[:]
        mu = jnp.mean(h, axis=0, keepdims=True)
        c = h - mu
        v = jnp.mean(c * c, axis=0, keepdims=True)
        h = g_ref[:] * c / jnp.sqrt(v + 1e-5) + be_ref[:]
        return jnp.maximum(h, 0.0)

    h = dense_bn_relu(x, w1_ref, b1_ref, g1_ref, be1_ref)
    h = dense_bn_relu(h, w2_ref, b2_ref, g2_ref, be2_ref)
    h = jnp.dot(h, w3_ref[:], preferred_element_type=jnp.float32) + b3_ref[:]
    out_ref[:] = jnp.max(h.reshape(M, K, 16), axis=1)


def _mlp(relx, rely, w1p, b1, g1, be1, w2, b2, g2, be2, w3, b3):
    args = (relx, rely, w1p, b1.reshape(1, -1), g1.reshape(1, -1),
            be1.reshape(1, -1), w2, b2.reshape(1, -1), g2.reshape(1, -1),
            be2.reshape(1, -1), w3, b3.reshape(1, -1))
    return pl.pallas_call(
        _mlp_body,
        out_shape=jax.ShapeDtypeStruct((M, 16), jnp.float32),
        interpret=_INTERPRET,
    )(*args)


def kernel(points, W1, b1, g1, be1, W2, b2, g2, be2, W3, b3):
    key = jax.random.key(42)
    init_idx = jax.random.randint(key, (1,), 0, N).astype(jnp.int32)

    ptx = points[:, 0]
    pty = points[:, 1]
    px2 = jnp.pad(ptx, (0, NPAD - N)).reshape(ROWS, 128)
    py2 = jnp.pad(pty, (0, NPAD - N)).reshape(ROWS, 128)
    cent = _fps(px2, py2, init_idx)

    px_col = px2.reshape(NPAD, 1)
    py_col = py2.reshape(NPAD, 1)
    cx_row = cent[:, 0].reshape(1, M)
    cy_row = cent[:, 1].reshape(1, M)
    nearest = _nearest(px_col, py_col, cx_row, cy_row)[:, 0]

    near_pad = jnp.where(jnp.arange(NPAD, dtype=jnp.int32) < N, nearest, M)
    relx_f, rely_f = _sc_group(near_pad, px2.reshape(NPAD), py2.reshape(NPAD),
                               cent[:, 0], cent[:, 1])
    relx = relx_f.reshape(M * K, 1)
    rely = rely_f.reshape(M * K, 1)

    perm = ([4 * fe + 2 * d for fe in range(NUM_FREQ) for d in range(2)]
            + [4 * fe + 2 * d + 1 for fe in range(NUM_FREQ) for d in range(2)])
    w1p = W1[jnp.asarray(perm), :]

    feat = _mlp(relx, rely, w1p, b1, g1, be1, W2, b2, g2, be2, W3, b3)
    return feat, cent


# speedup vs baseline: 1.1376x; 1.1376x over previous
import functools

import jax
import jax.numpy as jnp
import numpy as np
from jax import lax
from jax.experimental import pallas as pl
from jax.experimental.pallas import tpu as pltpu
from jax.experimental.pallas import tpu_sc as plsc

M = 512
K = 32
NUM_FREQ = 10
N = 20000
NPAD = 20480
ROWS = 160

_INTERPRET = False


def _fps_body(init_ref, px_ref, py_ref, cent_ref, dist_ref):
    px = px_ref[:]
    py = py_ref[:]
    row = lax.broadcasted_iota(jnp.int32, (ROWS, 128), 0)
    col = lax.broadcasted_iota(jnp.int32, (ROWS, 128), 1)
    fi = row * 128 + col
    valid = fi < N

    init = init_ref[0]
    m0 = fi == init
    cx0 = jnp.sum(jnp.where(m0, px, 0.0))
    cy0 = jnp.sum(jnp.where(m0, py, 0.0))
    cent_ref[0, 0] = cx0
    cent_ref[0, 1] = cy0
    dist_ref[:] = jnp.where(valid, jnp.inf, 0.0)

    def body(i, carry):
        cx, cy = carry
        dx = px - cx
        dy = py - cy
        nd2 = dx * dx + dy * dy + jnp.float32(1e-12)
        d2 = jnp.minimum(dist_ref[:], nd2)
        s = jnp.sqrt(d2)
        idx = jnp.argmax(s).astype(jnp.int32)
        m2 = fi == idx
        ncx = jnp.sum(jnp.where(m2, px, 0.0))
        ncy = jnp.sum(jnp.where(m2, py, 0.0))
        dist_ref[:] = jnp.where(m2, 0.0, d2)
        cent_ref[i, 0] = ncx
        cent_ref[i, 1] = ncy
        return (ncx, ncy)

    lax.fori_loop(1, M, body, (cx0, cy0))


def _fps(px, py, init):
    return pl.pallas_call(
        _fps_body,
        out_shape=jax.ShapeDtypeStruct((M, 2), jnp.float32),
        in_specs=[
            pl.BlockSpec(memory_space=pltpu.SMEM),
            pl.BlockSpec(memory_space=pltpu.VMEM),
            pl.BlockSpec(memory_space=pltpu.VMEM),
        ],
        out_specs=pl.BlockSpec(memory_space=pltpu.SMEM),
        scratch_shapes=[pltpu.VMEM((ROWS, 128), jnp.float32)],
        interpret=_INTERPRET,
    )(init, px, py)


_NB = 256


def _nn_body(px_ref, py_ref, cx_ref, cy_ref, out_ref):
    dx = px_ref[:] - cx_ref[:]
    dy = py_ref[:] - cy_ref[:]
    d2 = dx * dx + dy * dy
    m = jnp.min(d2, axis=1, keepdims=True)
    lane = lax.broadcasted_iota(jnp.int32, (_NB, M), 1)
    out_ref[:] = jnp.min(jnp.where(d2 == m, lane, M), axis=1, keepdims=True)


def _nearest(px_col, py_col, cx_row, cy_row):
    grid = NPAD // _NB
    return pl.pallas_call(
        _nn_body,
        grid=(grid,),
        out_shape=jax.ShapeDtypeStruct((NPAD, 1), jnp.int32),
        in_specs=[
            pl.BlockSpec((_NB, 1), lambda i: (i, 0)),
            pl.BlockSpec((_NB, 1), lambda i: (i, 0)),
            pl.BlockSpec((1, M), lambda i: (0, 0)),
            pl.BlockSpec((1, M), lambda i: (0, 0)),
        ],
        out_specs=pl.BlockSpec((_NB, 1), lambda i: (i, 0)),
        interpret=_INTERPRET,
    )(px_col, py_col, cx_row, cy_row)


_NW = 16
_CH = NPAD // _NW
_CPW = M // _NW
_SELPAD = M * K + 256


def _sc_group_body(near_hbm, ptx_hbm, pty_hbm, cx_hbm, cy_hbm,
                   relx_hbm, rely_hbm,
                   near_v, hist_v, allhist_v, start_v, cnts_v,
                   slot_v, val_v, selblk_v, ptx_v, pty_v,
                   cxl_v, cyl_v, relx_v, rely_v,
                   hist_sh, sel_sh, sem1, sem2):
    wid = lax.axis_index("s")
    base = wid * _CH
    cp1 = pltpu.async_copy(ptx_hbm, ptx_v, sem1)
    cp2 = pltpu.async_copy(pty_hbm, pty_v, sem2)
    pltpu.sync_copy(near_hbm.at[pl.ds(base, _CH)], near_v)
    pltpu.sync_copy(cx_hbm.at[pl.ds(wid * _CPW, _CPW)], cxl_v)
    pltpu.sync_copy(cy_hbm.at[pl.ds(wid * _CPW, _CPW)], cyl_v)

    zero16 = jnp.zeros((16,), jnp.int32)
    lane = lax.broadcasted_iota(jnp.int32, (16,), 0)
    mask0 = lane == 0
    for j in range(M // 16):
        hist_v[pl.ds(j * 16, 16)] = zero16

    def hbody(i, carry):
        c = plsc.load_gather(near_v, [zero16 + i])[0]
        valid = (c < M).astype(jnp.int32)
        cm16 = zero16 + jnp.minimum(c, M - 1)
        h = plsc.load_gather(hist_v, [cm16])[0]
        plsc.store_scatter(hist_v, [cm16], zero16 + h + valid, mask=mask0)
        return carry

    lax.fori_loop(0, _CH, hbody, 0)

    pltpu.sync_copy(hist_v, hist_sh.at[wid])
    plsc.subcore_barrier()
    pltpu.sync_copy(hist_sh, allhist_v)

    for j in range(M // 16):
        start_v[pl.ds(j * 16, 16)] = zero16
        cnts_v[pl.ds(j * 16, 16)] = zero16

    def pbody(t, carry):
        mlt = (t < wid).astype(jnp.int32)
        for j in range(M // 16):
            sl = pl.ds(j * 16, 16)
            v = allhist_v[t, sl]
            cnts_v[sl] = cnts_v[sl] + v
            start_v[sl] = start_v[sl] + v * mlt
        return carry

    lax.fori_loop(0, _NW, pbody, 0)

    dummy = M * K + 8 * wid
    dummy16 = zero16 + dummy
    for j in range(_CH // 128):
        for h in range(8):
            slot_v[j, pl.ds(h * 16, 16)] = dummy16
            val_v[j, pl.ds(h * 16, 16)] = zero16

    def bbody(i, w):
        c = plsc.load_gather(near_v, [zero16 + i])[0]
        valid = (c < M).astype(jnp.int32)
        cm = jnp.minimum(c, M - 1)
        cm16 = zero16 + cm
        r = plsc.load_gather(start_v, [cm16])[0]
        plsc.store_scatter(start_v, [cm16], zero16 + r + valid, mask=mask0)
        take = jnp.logical_and(valid == 1, r < K)
        takei = take.astype(jnp.int32)
        slot = jnp.where(take, cm * K + jnp.minimum(r, K - 1), dummy)
        val = jnp.where(take, base + i, 0)
        wr = zero16 + (w >> 7)
        wc = zero16 + (w & 127)
        plsc.store_scatter(slot_v, [wr, wc], zero16 + slot, mask=mask0)
        plsc.store_scatter(val_v, [wr, wc], zero16 + val, mask=mask0)
        return w + takei

    lax.fori_loop(0, _CH, bbody, 0)

    for j in range(_CH // 128):
        pltpu.sync_copy(val_v.at[j], sel_sh.at[slot_v.at[j]])
    plsc.subcore_barrier()

    pltpu.sync_copy(sel_sh.at[pl.ds(wid * _CPW * K, _CPW * K)], selblk_v)
    cp1.wait()
    cp2.wait()

    cxl0 = cxl_v[pl.ds(0, 16)]
    cxl1 = cxl_v[pl.ds(16, 16)]
    cyl0 = cyl_v[pl.ds(0, 16)]
    cyl1 = cyl_v[pl.ds(16, 16)]
    for cl in range(_CPW):
        cnt = plsc.load_gather(cnts_v, [zero16 + wid * _CPW + cl])[0]
        cntc = jnp.minimum(cnt, K)
        last = plsc.load_gather(
            selblk_v, [zero16 + cl * K + jnp.maximum(cntc - 1, 0)])[0]
        cx = cxl0[cl] if cl < 16 else cxl1[cl - 16]
        cy = cyl0[cl] if cl < 16 else cyl1[cl - 16]
        emptyc = cnt == 0
        for j in range(K // 16):
            off = pl.ds(cl * K + j * 16, 16)
            kvec = lane + j * 16
            s = jnp.where(kvec < cntc, selblk_v[off], last)
            s = jnp.clip(s, 0, N - 1)
            gx = plsc.load_gather(ptx_v, [s])
            gy = plsc.load_gather(pty_v, [s])
            first = jnp.logical_and(emptyc, kvec == 0)
            rx = jnp.where(first, 0.0, jnp.where(emptyc, 0.0 - cx, gx - cx))
            ry = jnp.where(first, 0.0, jnp.where(emptyc, 0.0 - cy, gy - cy))
            relx_v[off] = rx
            rely_v[off] = ry

    pltpu.sync_copy(relx_v, relx_hbm.at[pl.ds(wid * _CPW * K, _CPW * K)])
    pltpu.sync_copy(rely_v, rely_hbm.at[pl.ds(wid * _CPW * K, _CPW * K)])


def _sc_group(near_pad, ptx_pad, pty_pad, centx, centy):
    mesh = plsc.VectorSubcoreMesh(core_axis_name="c", subcore_axis_name="s",
                                  num_cores=1)
    return pl.kernel(
        _sc_group_body,
        out_type=[jax.ShapeDtypeStruct((M * K,), jnp.float32),
                  jax.ShapeDtypeStruct((M * K,), jnp.float32)],
        mesh=mesh,
        compiler_params=pltpu.CompilerParams(needs_layout_passes=False),
        scratch_types=[
            pltpu.VMEM((_CH,), jnp.int32),
            pltpu.VMEM((M,), jnp.int32),
            pltpu.VMEM((_NW, M), jnp.int32),
            pltpu.VMEM((M,), jnp.int32),
            pltpu.VMEM((M,), jnp.int32),
            pltpu.VMEM((_CH // 128, 128), jnp.int32),
            pltpu.VMEM((_CH // 128, 128), jnp.int32),
            pltpu.VMEM((_CPW * K,), jnp.int32),
            pltpu.VMEM((NPAD,), jnp.float32),
            pltpu.VMEM((NPAD,), jnp.float32),
            pltpu.VMEM((_CPW,), jnp.float32),
            pltpu.VMEM((_CPW,), jnp.float32),
            pltpu.VMEM((_CPW * K,), jnp.float32),
            pltpu.VMEM((_CPW * K,), jnp.float32),
            pltpu.VMEM_SHARED((_NW, M), jnp.int32),
            pltpu.VMEM_SHARED((_SELPAD,), jnp.int32),
            pltpu.SemaphoreType.DMA,
            pltpu.SemaphoreType.DMA,
        ],
    )(near_pad, ptx_pad, pty_pad, centx, centy)


def _mlp_body(relx_ref, rely_ref, w1_ref, b1_ref, g1_ref, be1_ref,
              w2_ref, b2_ref, g2_ref, be2_ref, w3_ref, b3_ref, out_ref):
    rx = relx_ref[:]
    ry = rely_ref[:]
    ii = lax.broadcasted_iota(jnp.int32, (1, 2 * NUM_FREQ), 1)
    freq = lax.shift_left(1, ii // 2).astype(jnp.float32) * jnp.float32(np.pi)
    rsel = jnp.where((ii % 2) == 0, rx, ry)
    args = rsel * freq
    x = jnp.concatenate([jnp.sin(args), jnp.cos(args)], axis=1)

    def dense_bn_relu(h, w_ref, b_ref, g_ref, be_ref):
        h = jnp.dot(h, w_ref[:], preferred_element_type=jnp.float32) + b_ref[:]
        mu = jnp.mean(h, axis=0, keepdims=True)
        c = h - mu
        v = jnp.mean(c * c, axis=0, keepdims=True)
        h = g_ref[:] * c / jnp.sqrt(v + 1e-5) + be_ref[:]
        return jnp.maximum(h, 0.0)

    h = dense_bn_relu(x, w1_ref, b1_ref, g1_ref, be1_ref)
    h = dense_bn_relu(h, w2_ref, b2_ref, g2_ref, be2_ref)
    h = jnp.dot(h, w3_ref[:], preferred_element_type=jnp.float32) + b3_ref[:]
    out_ref[:] = jnp.max(h.reshape(M, K, 16), axis=1)


def _mlp(relx, rely, w1p, b1, g1, be1, w2, b2, g2, be2, w3, b3):
    args = (relx, rely, w1p, b1.reshape(1, -1), g1.reshape(1, -1),
            be1.reshape(1, -1), w2, b2.reshape(1, -1), g2.reshape(1, -1),
            be2.reshape(1, -1), w3, b3.reshape(1, -1))
    return pl.pallas_call(
        _mlp_body,
        out_shape=jax.ShapeDtypeStruct((M, 16), jnp.float32),
        interpret=_INTERPRET,
    )(*args)


def kernel(points, W1, b1, g1, be1, W2, b2, g2, be2, W3, b3):
    key = jax.random.key(42)
    init_idx = jax.random.randint(key, (1,), 0, N).astype(jnp.int32)

    ptx = points[:, 0]
    pty = points[:, 1]
    px2 = jnp.pad(ptx, (0, NPAD - N)).reshape(ROWS, 128)
    py2 = jnp.pad(pty, (0, NPAD - N)).reshape(ROWS, 128)
    cent = _fps(px2, py2, init_idx)

    px_col = px2.reshape(NPAD, 1)
    py_col = py2.reshape(NPAD, 1)
    cx_row = cent[:, 0].reshape(1, M)
    cy_row = cent[:, 1].reshape(1, M)
    nearest = _nearest(px_col, py_col, cx_row, cy_row)[:, 0]

    near_pad = jnp.where(jnp.arange(NPAD, dtype=jnp.int32) < N, nearest, M)
    relx_f, rely_f = _sc_group(near_pad, px2.reshape(NPAD), py2.reshape(NPAD),
                               cent[:, 0], cent[:, 1])
    relx = relx_f.reshape(M * K, 1)
    rely = rely_f.reshape(M * K, 1)

    perm = ([4 * fe + 2 * d for fe in range(NUM_FREQ) for d in range(2)]
            + [4 * fe + 2 * d + 1 for fe in range(NUM_FREQ) for d in range(2)])
    w1p = W1[jnp.asarray(perm), :]

    feat = _mlp(relx, rely, w1p, b1, g1, be1, W2, b2, g2, be2, W3, b3)
    return feat, cent


# speedup vs baseline: 1.3042x; 1.1464x over previous
import functools

import jax
import jax.numpy as jnp
import numpy as np
from jax import lax
from jax.experimental import pallas as pl
from jax.experimental.pallas import tpu as pltpu
from jax.experimental.pallas import tpu_sc as plsc

M = 512
K = 32
NUM_FREQ = 10
N = 20000
NPAD = 20480
ROWS = 160

_INTERPRET = False


def _fps_body(init_ref, px_ref, py_ref, cent_ref, dist_ref):
    px = px_ref[:]
    py = py_ref[:]
    row = lax.broadcasted_iota(jnp.int32, (ROWS, 128), 0)
    col = lax.broadcasted_iota(jnp.int32, (ROWS, 128), 1)
    fi = row * 128 + col
    valid = fi < N

    init = init_ref[0]
    m0 = fi == init
    cx0 = jnp.sum(jnp.where(m0, px, 0.0))
    cy0 = jnp.sum(jnp.where(m0, py, 0.0))
    cent_ref[0, 0] = cx0
    cent_ref[0, 1] = cy0
    dist_ref[:] = jnp.where(valid, jnp.inf, 0.0)

    def body(i, carry):
        cx, cy = carry
        dx = px - cx
        dy = py - cy
        nd2 = dx * dx + dy * dy + jnp.float32(1e-12)
        d2 = jnp.minimum(dist_ref[:], nd2)
        s = jnp.sqrt(d2)
        idx = jnp.argmax(s).astype(jnp.int32)
        m2 = fi == idx
        ncx = jnp.sum(jnp.where(m2, px, 0.0))
        ncy = jnp.sum(jnp.where(m2, py, 0.0))
        dist_ref[:] = jnp.where(m2, 0.0, d2)
        cent_ref[i, 0] = ncx
        cent_ref[i, 1] = ncy
        return (ncx, ncy)

    lax.fori_loop(1, M, body, (cx0, cy0))


def _fps(px, py, init):
    return pl.pallas_call(
        _fps_body,
        out_shape=jax.ShapeDtypeStruct((M, 2), jnp.float32),
        in_specs=[
            pl.BlockSpec(memory_space=pltpu.SMEM),
            pl.BlockSpec(memory_space=pltpu.VMEM),
            pl.BlockSpec(memory_space=pltpu.VMEM),
        ],
        out_specs=pl.BlockSpec(memory_space=pltpu.SMEM),
        scratch_shapes=[pltpu.VMEM((ROWS, 128), jnp.float32)],
        interpret=_INTERPRET,
    )(init, px, py)


_NB = 256


def _nn_body(px_ref, py_ref, cx_ref, cy_ref, out_ref):
    dx = px_ref[:] - cx_ref[:]
    dy = py_ref[:] - cy_ref[:]
    d2 = dx * dx + dy * dy
    m = jnp.min(d2, axis=1, keepdims=True)
    lane = lax.broadcasted_iota(jnp.int32, (_NB, M), 1)
    out_ref[:] = jnp.min(jnp.where(d2 == m, lane, M), axis=1, keepdims=True)


def _nearest(px_col, py_col, cx_row, cy_row):
    grid = NPAD // _NB
    return pl.pallas_call(
        _nn_body,
        grid=(grid,),
        out_shape=jax.ShapeDtypeStruct((NPAD, 1), jnp.int32),
        in_specs=[
            pl.BlockSpec((_NB, 1), lambda i: (i, 0)),
            pl.BlockSpec((_NB, 1), lambda i: (i, 0)),
            pl.BlockSpec((1, M), lambda i: (0, 0)),
            pl.BlockSpec((1, M), lambda i: (0, 0)),
        ],
        out_specs=pl.BlockSpec((_NB, 1), lambda i: (i, 0)),
        interpret=_INTERPRET,
    )(px_col, py_col, cx_row, cy_row)


_NW = 16
_CH = NPAD // _NW
_CPW = M // _NW
_SELPAD = M * K + 256


def _sc_group_body(near_hbm, ptx_hbm, pty_hbm, cx_hbm, cy_hbm,
                   relx_hbm, rely_hbm,
                   near_v, hist_v, allhist_v, start_v, cnts_v,
                   sel_local, merge_v, selblk_v, ptx_v, pty_v,
                   cxl_v, cyl_v, relx_v, rely_v,
                   hist_sh, sel_sh, sem1, sem2):
    wid = lax.axis_index("s")
    base = wid * _CH
    cp1 = pltpu.async_copy(ptx_hbm, ptx_v, sem1)
    cp2 = pltpu.async_copy(pty_hbm, pty_v, sem2)
    pltpu.sync_copy(near_hbm.at[pl.ds(base, _CH)], near_v)
    pltpu.sync_copy(cx_hbm.at[pl.ds(wid * _CPW, _CPW)], cxl_v)
    pltpu.sync_copy(cy_hbm.at[pl.ds(wid * _CPW, _CPW)], cyl_v)

    zero16 = jnp.zeros((16,), jnp.int32)
    lane = lax.broadcasted_iota(jnp.int32, (16,), 0)
    for j in range(M // 16):
        hist_v[pl.ds(j * 16, 16)] = zero16

    def habody(t, carry):
        c16 = near_v[pl.ds(t * 16, 16)]
        validm = c16 < M
        cm16 = jnp.minimum(c16, M - 1)
        cnt16, lastm = plsc.scan_count(cm16, validm)
        h16 = plsc.load_gather(hist_v, [cm16])
        plsc.store_scatter(hist_v, [cm16], h16 + cnt16, mask=lastm)
        return carry

    lax.fori_loop(0, _CH // 16, habody, 0)

    pltpu.sync_copy(hist_v, hist_sh.at[wid])
    plsc.subcore_barrier()
    pltpu.sync_copy(hist_sh, allhist_v)

    for j in range(M // 16):
        start_v[pl.ds(j * 16, 16)] = zero16
        cnts_v[pl.ds(j * 16, 16)] = zero16

    def pbody(t, carry):
        mlt = (t < wid).astype(jnp.int32)
        for j in range(M // 16):
            sl = pl.ds(j * 16, 16)
            v = allhist_v[t, sl]
            cnts_v[sl] = cnts_v[sl] + v
            start_v[sl] = start_v[sl] + v * mlt
        return carry

    lax.fori_loop(0, _NW, pbody, 0)

    def zbody(t, carry):
        for h in range(8):
            sel_local[pl.ds(t * 128 + h * 16, 16)] = zero16
        return carry

    lax.fori_loop(0, M * K // 128, zbody, 0)

    def bbody(t, carry):
        c16 = near_v[pl.ds(t * 16, 16)]
        validm = c16 < M
        cm16 = jnp.minimum(c16, M - 1)
        cnt16, lastm = plsc.scan_count(cm16, validm)
        s16 = plsc.load_gather(start_v, [cm16])
        r16 = s16 + cnt16 - 1
        plsc.store_scatter(start_v, [cm16], s16 + cnt16, mask=lastm)
        take = jnp.logical_and(validm, r16 < K)
        slot16 = cm16 * K + jnp.minimum(r16, K - 1)
        val16 = base + t * 16 + lane
        plsc.store_scatter(sel_local, [slot16], val16, mask=take)
        return carry

    lax.fori_loop(0, _CH // 16, bbody, 0)

    pltpu.sync_copy(sel_local, sel_sh.at[wid])
    plsc.subcore_barrier()

    pltpu.sync_copy(sel_sh.at[:, pl.ds(wid * _CPW * K, _CPW * K)], merge_v)

    def mbody(j, carry):
        acc = merge_v[0, pl.ds(j * 16, 16)]
        for t in range(1, _NW):
            acc = acc + merge_v[t, pl.ds(j * 16, 16)]
        selblk_v[pl.ds(j * 16, 16)] = acc
        return carry

    lax.fori_loop(0, _CPW * K // 16, mbody, 0)
    cp1.wait()
    cp2.wait()

    cxl0 = cxl_v[pl.ds(0, 16)]
    cxl1 = cxl_v[pl.ds(16, 16)]
    cyl0 = cyl_v[pl.ds(0, 16)]
    cyl1 = cyl_v[pl.ds(16, 16)]
    for cl in range(_CPW):
        cnt = plsc.load_gather(cnts_v, [zero16 + wid * _CPW + cl])[0]
        cntc = jnp.minimum(cnt, K)
        last = plsc.load_gather(
            selblk_v, [zero16 + cl * K + jnp.maximum(cntc - 1, 0)])[0]
        cx = cxl0[cl] if cl < 16 else cxl1[cl - 16]
        cy = cyl0[cl] if cl < 16 else cyl1[cl - 16]
        emptyc = cnt == 0
        for j in range(K // 16):
            off = pl.ds(cl * K + j * 16, 16)
            kvec = lane + j * 16
            s = jnp.where(kvec < cntc, selblk_v[off], last)
            s = jnp.clip(s, 0, N - 1)
            gx = plsc.load_gather(ptx_v, [s])
            gy = plsc.load_gather(pty_v, [s])
            first = jnp.logical_and(emptyc, kvec == 0)
            rx = jnp.where(first, 0.0, jnp.where(emptyc, 0.0 - cx, gx - cx))
            ry = jnp.where(first, 0.0, jnp.where(emptyc, 0.0 - cy, gy - cy))
            relx_v[off] = rx
            rely_v[off] = ry

    pltpu.sync_copy(relx_v, relx_hbm.at[pl.ds(wid * _CPW * K, _CPW * K)])
    pltpu.sync_copy(rely_v, rely_hbm.at[pl.ds(wid * _CPW * K, _CPW * K)])


def _sc_group(near_pad, ptx_pad, pty_pad, centx, centy):
    mesh = plsc.VectorSubcoreMesh(core_axis_name="c", subcore_axis_name="s",
                                  num_cores=1)
    return pl.kernel(
        _sc_group_body,
        out_type=[jax.ShapeDtypeStruct((M * K,), jnp.float32),
                  jax.ShapeDtypeStruct((M * K,), jnp.float32)],
        mesh=mesh,
        compiler_params=pltpu.CompilerParams(needs_layout_passes=False),
        scratch_types=[
            pltpu.VMEM((_CH,), jnp.int32),
            pltpu.VMEM((M,), jnp.int32),
            pltpu.VMEM((_NW, M), jnp.int32),
            pltpu.VMEM((M,), jnp.int32),
            pltpu.VMEM((M,), jnp.int32),
            pltpu.VMEM((M * K,), jnp.int32),
            pltpu.VMEM((_NW, _CPW * K), jnp.int32),
            pltpu.VMEM((_CPW * K,), jnp.int32),
            pltpu.VMEM((NPAD,), jnp.float32),
            pltpu.VMEM((NPAD,), jnp.float32),
            pltpu.VMEM((_CPW,), jnp.float32),
            pltpu.VMEM((_CPW,), jnp.float32),
            pltpu.VMEM((_CPW * K,), jnp.float32),
            pltpu.VMEM((_CPW * K,), jnp.float32),
            pltpu.VMEM_SHARED((_NW, M), jnp.int32),
            pltpu.VMEM_SHARED((_NW, M * K), jnp.int32),
            pltpu.SemaphoreType.DMA,
            pltpu.SemaphoreType.DMA,
        ],
    )(near_pad, ptx_pad, pty_pad, centx, centy)


def _mlp_body(relx_ref, rely_ref, w1_ref, b1_ref, g1_ref, be1_ref,
              w2_ref, b2_ref, g2_ref, be2_ref, w3_ref, b3_ref, out_ref):
    rx = relx_ref[:]
    ry = rely_ref[:]
    ii = lax.broadcasted_iota(jnp.int32, (1, 2 * NUM_FREQ), 1)
    freq = lax.shift_left(1, ii // 2).astype(jnp.float32) * jnp.float32(np.pi)
    rsel = jnp.where((ii % 2) == 0, rx, ry)
    args = rsel * freq
    x = jnp.concatenate([jnp.sin(args), jnp.cos(args)], axis=1)

    def dense_bn_relu(h, w_ref, b_ref, g_ref, be_ref):
        h = jnp.dot(h, w_ref[:], preferred_element_type=jnp.float32) + b_ref[:]
        mu = jnp.mean(h, axis=0, keepdims=True)
        c = h - mu
        v = jnp.mean(c * c, axis=0, keepdims=True)
        h = g_ref[:] * c / jnp.sqrt(v + 1e-5) + be_ref[:]
        return jnp.maximum(h, 0.0)

    h = dense_bn_relu(x, w1_ref, b1_ref, g1_ref, be1_ref)
    h = dense_bn_relu(h, w2_ref, b2_ref, g2_ref, be2_ref)
    h = jnp.dot(h, w3_ref[:], preferred_element_type=jnp.float32) + b3_ref[:]
    out_ref[:] = jnp.max(h.reshape(M, K, 16), axis=1)


def _mlp(relx, rely, w1p, b1, g1, be1, w2, b2, g2, be2, w3, b3):
    args = (relx, rely, w1p, b1.reshape(1, -1), g1.reshape(1, -1),
            be1.reshape(1, -1), w2, b2.reshape(1, -1), g2.reshape(1, -1),
            be2.reshape(1, -1), w3, b3.reshape(1, -1))
    return pl.pallas_call(
        _mlp_body,
        out_shape=jax.ShapeDtypeStruct((M, 16), jnp.float32),
        interpret=_INTERPRET,
    )(*args)


def kernel(points, W1, b1, g1, be1, W2, b2, g2, be2, W3, b3):
    key = jax.random.key(42)
    init_idx = jax.random.randint(key, (1,), 0, N).astype(jnp.int32)

    ptx = points[:, 0]
    pty = points[:, 1]
    px2 = jnp.pad(ptx, (0, NPAD - N)).reshape(ROWS, 128)
    py2 = jnp.pad(pty, (0, NPAD - N)).reshape(ROWS, 128)
    cent = _fps(px2, py2, init_idx)

    px_col = px2.reshape(NPAD, 1)
    py_col = py2.reshape(NPAD, 1)
    cx_row = cent[:, 0].reshape(1, M)
    cy_row = cent[:, 1].reshape(1, M)
    nearest = _nearest(px_col, py_col, cx_row, cy_row)[:, 0]

    near_pad = jnp.where(jnp.arange(NPAD, dtype=jnp.int32) < N, nearest, M)
    relx_f, rely_f = _sc_group(near_pad, px2.reshape(NPAD), py2.reshape(NPAD),
                               cent[:, 0], cent[:, 1])
    relx = relx_f.reshape(M * K, 1)
    rely = rely_f.reshape(M * K, 1)

    perm = ([4 * fe + 2 * d for fe in range(NUM_FREQ) for d in range(2)]
            + [4 * fe + 2 * d + 1 for fe in range(NUM_FREQ) for d in range(2)])
    w1p = W1[jnp.asarray(perm), :]

    feat = _mlp(relx, rely, w1p, b1, g1, be1, W2, b2, g2, be2, W3, b3)
    return feat, cent


# speedup vs baseline: 1.5511x; 1.1892x over previous
import functools

import jax
import jax.numpy as jnp
import numpy as np
from jax import lax
from jax.experimental import pallas as pl
from jax.experimental.pallas import tpu as pltpu
from jax.experimental.pallas import tpu_sc as plsc

M = 512
K = 32
NUM_FREQ = 10
N = 20000
NPAD = 20480
ROWS = 160

_INTERPRET = False


def _fps_body(init_ref, pxs_ref, pys_ref, px_ref, py_ref, cent_ref, dist_ref):
    px = px_ref[:]
    py = py_ref[:]
    row = lax.broadcasted_iota(jnp.int32, (ROWS, 128), 0)
    col = lax.broadcasted_iota(jnp.int32, (ROWS, 128), 1)
    fi = row * 128 + col
    valid = fi < N

    init = init_ref[0]
    cx0 = pxs_ref[init]
    cy0 = pys_ref[init]
    cent_ref[0, 0] = cx0
    cent_ref[0, 1] = cy0
    dist_ref[:] = jnp.where(valid, jnp.inf, 0.0)

    def body(i, carry):
        cx, cy = carry
        dx = px - cx
        dy = py - cy
        nd2 = dx * dx + dy * dy + jnp.float32(1e-12)
        d2 = jnp.minimum(dist_ref[:], nd2)
        s = jnp.sqrt(d2)
        mx = jnp.max(s)
        idx = jnp.min(jnp.where(s == mx, fi, NPAD))
        m2 = fi == idx
        dist_ref[:] = jnp.where(m2, 0.0, d2)
        ncx = pxs_ref[idx]
        ncy = pys_ref[idx]
        cent_ref[i, 0] = ncx
        cent_ref[i, 1] = ncy
        return (ncx, ncy)

    lax.fori_loop(1, M, body, (cx0, cy0))


def _fps(px, py, pxflat, pyflat, init):
    return pl.pallas_call(
        _fps_body,
        out_shape=jax.ShapeDtypeStruct((M, 2), jnp.float32),
        in_specs=[
            pl.BlockSpec(memory_space=pltpu.SMEM),
            pl.BlockSpec(memory_space=pltpu.SMEM),
            pl.BlockSpec(memory_space=pltpu.SMEM),
            pl.BlockSpec(memory_space=pltpu.VMEM),
            pl.BlockSpec(memory_space=pltpu.VMEM),
        ],
        out_specs=pl.BlockSpec(memory_space=pltpu.SMEM),
        scratch_shapes=[pltpu.VMEM((ROWS, 128), jnp.float32)],
        interpret=_INTERPRET,
    )(init, pxflat, pyflat, px, py)


_NB = 256


def _nn_body(px_ref, py_ref, cx_ref, cy_ref, out_ref):
    dx = px_ref[:] - cx_ref[:]
    dy = py_ref[:] - cy_ref[:]
    d2 = dx * dx + dy * dy
    m = jnp.min(d2, axis=1, keepdims=True)
    lane = lax.broadcasted_iota(jnp.int32, (_NB, M), 1)
    out_ref[:] = jnp.min(jnp.where(d2 == m, lane, M), axis=1, keepdims=True)


def _nearest(px_col, py_col, cx_row, cy_row):
    grid = NPAD // _NB
    return pl.pallas_call(
        _nn_body,
        grid=(grid,),
        out_shape=jax.ShapeDtypeStruct((NPAD, 1), jnp.int32),
        in_specs=[
            pl.BlockSpec((_NB, 1), lambda i: (i, 0)),
            pl.BlockSpec((_NB, 1), lambda i: (i, 0)),
            pl.BlockSpec((1, M), lambda i: (0, 0)),
            pl.BlockSpec((1, M), lambda i: (0, 0)),
        ],
        out_specs=pl.BlockSpec((_NB, 1), lambda i: (i, 0)),
        interpret=_INTERPRET,
    )(px_col, py_col, cx_row, cy_row)


_NW = 16
_CH = NPAD // _NW
_CPW = M // _NW
_SELPAD = M * K + 256


def _sc_group_body(near_hbm, ptx_hbm, pty_hbm, cx_hbm, cy_hbm,
                   relx_hbm, rely_hbm,
                   near_v, hist_v, allhist_v, start_v, cnts_v,
                   sel_local, merge_v, selblk_v, ptx_v, pty_v,
                   cxl_v, cyl_v, relx_v, rely_v,
                   hist_sh, sel_sh, sem1, sem2):
    wid = lax.axis_index("s")
    base = wid * _CH
    cp1 = pltpu.async_copy(ptx_hbm, ptx_v, sem1)
    cp2 = pltpu.async_copy(pty_hbm, pty_v, sem2)
    pltpu.sync_copy(near_hbm.at[pl.ds(base, _CH)], near_v)
    pltpu.sync_copy(cx_hbm.at[pl.ds(wid * _CPW, _CPW)], cxl_v)
    pltpu.sync_copy(cy_hbm.at[pl.ds(wid * _CPW, _CPW)], cyl_v)

    zero16 = jnp.zeros((16,), jnp.int32)
    lane = lax.broadcasted_iota(jnp.int32, (16,), 0)
    for j in range(M // 16):
        hist_v[pl.ds(j * 16, 16)] = zero16

    def habody(t, carry):
        c16 = near_v[pl.ds(t * 16, 16)]
        validm = c16 < M
        cm16 = jnp.minimum(c16, M - 1)
        cnt16, lastm = plsc.scan_count(cm16, validm)
        h16 = plsc.load_gather(hist_v, [cm16])
        plsc.store_scatter(hist_v, [cm16], h16 + cnt16, mask=lastm)
        return carry

    lax.fori_loop(0, _CH // 16, habody, 0)

    pltpu.sync_copy(hist_v, hist_sh.at[wid])
    plsc.subcore_barrier()
    pltpu.sync_copy(hist_sh, allhist_v)

    for j in range(M // 16):
        start_v[pl.ds(j * 16, 16)] = zero16
        cnts_v[pl.ds(j * 16, 16)] = zero16

    def pbody(t, carry):
        mlt = (t < wid).astype(jnp.int32)
        for j in range(M // 16):
            sl = pl.ds(j * 16, 16)
            v = allhist_v[t, sl]
            cnts_v[sl] = cnts_v[sl] + v
            start_v[sl] = start_v[sl] + v * mlt
        return carry

    lax.fori_loop(0, _NW, pbody, 0)

    def zbody(t, carry):
        for h in range(8):
            sel_local[pl.ds(t * 128 + h * 16, 16)] = zero16
        return carry

    lax.fori_loop(0, M * K // 128, zbody, 0)

    def bbody(t, carry):
        c16 = near_v[pl.ds(t * 16, 16)]
        validm = c16 < M
        cm16 = jnp.minimum(c16, M - 1)
        cnt16, lastm = plsc.scan_count(cm16, validm)
        s16 = plsc.load_gather(start_v, [cm16])
        r16 = s16 + cnt16 - 1
        plsc.store_scatter(start_v, [cm16], s16 + cnt16, mask=lastm)
        take = jnp.logical_and(validm, r16 < K)
        slot16 = cm16 * K + jnp.minimum(r16, K - 1)
        val16 = base + t * 16 + lane
        plsc.store_scatter(sel_local, [slot16], val16, mask=take)
        return carry

    lax.fori_loop(0, _CH // 16, bbody, 0)

    pltpu.sync_copy(sel_local, sel_sh.at[wid])
    plsc.subcore_barrier()

    pltpu.sync_copy(sel_sh.at[:, pl.ds(wid * _CPW * K, _CPW * K)], merge_v)

    def mbody(j, carry):
        acc = merge_v[0, pl.ds(j * 16, 16)]
        for t in range(1, _NW):
            acc = acc + merge_v[t, pl.ds(j * 16, 16)]
        selblk_v[pl.ds(j * 16, 16)] = acc
        return carry

    lax.fori_loop(0, _CPW * K // 16, mbody, 0)
    cp1.wait()
    cp2.wait()

    cxl0 = cxl_v[pl.ds(0, 16)]
    cxl1 = cxl_v[pl.ds(16, 16)]
    cyl0 = cyl_v[pl.ds(0, 16)]
    cyl1 = cyl_v[pl.ds(16, 16)]
    for cl in range(_CPW):
        cnt = plsc.load_gather(cnts_v, [zero16 + wid * _CPW + cl])[0]
        cntc = jnp.minimum(cnt, K)
        last = plsc.load_gather(
            selblk_v, [zero16 + cl * K + jnp.maximum(cntc - 1, 0)])[0]
        cx = cxl0[cl] if cl < 16 else cxl1[cl - 16]
        cy = cyl0[cl] if cl < 16 else cyl1[cl - 16]
        emptyc = cnt == 0
        for j in range(K // 16):
            off = pl.ds(cl * K + j * 16, 16)
            kvec = lane + j * 16
            s = jnp.where(kvec < cntc, selblk_v[off], last)
            s = jnp.clip(s, 0, N - 1)
            gx = plsc.load_gather(ptx_v, [s])
            gy = plsc.load_gather(pty_v, [s])
            first = jnp.logical_and(emptyc, kvec == 0)
            rx = jnp.where(first, 0.0, jnp.where(emptyc, 0.0 - cx, gx - cx))
            ry = jnp.where(first, 0.0, jnp.where(emptyc, 0.0 - cy, gy - cy))
            relx_v[off] = rx
            rely_v[off] = ry

    pltpu.sync_copy(relx_v, relx_hbm.at[pl.ds(wid * _CPW * K, _CPW * K)])
    pltpu.sync_copy(rely_v, rely_hbm.at[pl.ds(wid * _CPW * K, _CPW * K)])


def _sc_group(near_pad, ptx_pad, pty_pad, centx, centy):
    mesh = plsc.VectorSubcoreMesh(core_axis_name="c", subcore_axis_name="s",
                                  num_cores=1)
    return pl.kernel(
        _sc_group_body,
        out_type=[jax.ShapeDtypeStruct((M * K,), jnp.float32),
                  jax.ShapeDtypeStruct((M * K,), jnp.float32)],
        mesh=mesh,
        compiler_params=pltpu.CompilerParams(needs_layout_passes=False),
        scratch_types=[
            pltpu.VMEM((_CH,), jnp.int32),
            pltpu.VMEM((M,), jnp.int32),
            pltpu.VMEM((_NW, M), jnp.int32),
            pltpu.VMEM((M,), jnp.int32),
            pltpu.VMEM((M,), jnp.int32),
            pltpu.VMEM((M * K,), jnp.int32),
            pltpu.VMEM((_NW, _CPW * K), jnp.int32),
            pltpu.VMEM((_CPW * K,), jnp.int32),
            pltpu.VMEM((NPAD,), jnp.float32),
            pltpu.VMEM((NPAD,), jnp.float32),
            pltpu.VMEM((_CPW,), jnp.float32),
            pltpu.VMEM((_CPW,), jnp.float32),
            pltpu.VMEM((_CPW * K,), jnp.float32),
            pltpu.VMEM((_CPW * K,), jnp.float32),
            pltpu.VMEM_SHARED((_NW, M), jnp.int32),
            pltpu.VMEM_SHARED((_NW, M * K), jnp.int32),
            pltpu.SemaphoreType.DMA,
            pltpu.SemaphoreType.DMA,
        ],
    )(near_pad, ptx_pad, pty_pad, centx, centy)


def _mlp_body(relx_ref, rely_ref, w1_ref, b1_ref, g1_ref, be1_ref,
              w2_ref, b2_ref, g2_ref, be2_ref, w3_ref, b3_ref, out_ref):
    rx = relx_ref[:]
    ry = rely_ref[:]
    ii = lax.broadcasted_iota(jnp.int32, (1, 2 * NUM_FREQ), 1)
    freq = lax.shift_left(1, ii // 2).astype(jnp.float32) * jnp.float32(np.pi)
    rsel = jnp.where((ii % 2) == 0, rx, ry)
    args = rsel * freq
    x = jnp.concatenate([jnp.sin(args), jnp.cos(args)], axis=1)

    def dense_bn_relu(h, w_ref, b_ref, g_ref, be_ref):
        h = jnp.dot(h, w_ref[:], preferred_element_type=jnp.float32) + b_ref[:]
        mu = jnp.mean(h, axis=0, keepdims=True)
        c = h - mu
        v = jnp.mean(c * c, axis=0, keepdims=True)
        h = g_ref[:] * c / jnp.sqrt(v + 1e-5) + be_ref[:]
        return jnp.maximum(h, 0.0)

    h = dense_bn_relu(x, w1_ref, b1_ref, g1_ref, be1_ref)
    h = dense_bn_relu(h, w2_ref, b2_ref, g2_ref, be2_ref)
    h = jnp.dot(h, w3_ref[:], preferred_element_type=jnp.float32) + b3_ref[:]
    out_ref[:] = jnp.max(h.reshape(M, K, 16), axis=1)


def _mlp(relx, rely, w1p, b1, g1, be1, w2, b2, g2, be2, w3, b3):
    args = (relx, rely, w1p, b1.reshape(1, -1), g1.reshape(1, -1),
            be1.reshape(1, -1), w2, b2.reshape(1, -1), g2.reshape(1, -1),
            be2.reshape(1, -1), w3, b3.reshape(1, -1))
    return pl.pallas_call(
        _mlp_body,
        out_shape=jax.ShapeDtypeStruct((M, 16), jnp.float32),
        interpret=_INTERPRET,
    )(*args)


def kernel(points, W1, b1, g1, be1, W2, b2, g2, be2, W3, b3):
    key = jax.random.key(42)
    init_idx = jax.random.randint(key, (1,), 0, N).astype(jnp.int32)

    ptx = points[:, 0]
    pty = points[:, 1]
    px2 = jnp.pad(ptx, (0, NPAD - N)).reshape(ROWS, 128)
    py2 = jnp.pad(pty, (0, NPAD - N)).reshape(ROWS, 128)
    cent = _fps(px2, py2, px2.reshape(NPAD), py2.reshape(NPAD),
                init_idx)

    px_col = px2.reshape(NPAD, 1)
    py_col = py2.reshape(NPAD, 1)
    cx_row = cent[:, 0].reshape(1, M)
    cy_row = cent[:, 1].reshape(1, M)
    nearest = _nearest(px_col, py_col, cx_row, cy_row)[:, 0]

    near_pad = jnp.where(jnp.arange(NPAD, dtype=jnp.int32) < N, nearest, M)
    relx_f, rely_f = _sc_group(near_pad, px2.reshape(NPAD), py2.reshape(NPAD),
                               cent[:, 0], cent[:, 1])
    relx = relx_f.reshape(M * K, 1)
    rely = rely_f.reshape(M * K, 1)

    perm = ([4 * fe + 2 * d for fe in range(NUM_FREQ) for d in range(2)]
            + [4 * fe + 2 * d + 1 for fe in range(NUM_FREQ) for d in range(2)])
    w1p = W1[jnp.asarray(perm), :]

    feat = _mlp(relx, rely, w1p, b1, g1, be1, W2, b2, g2, be2, W3, b3)
    return feat, cent


# speedup vs baseline: 1.6840x; 1.0857x over previous
import functools

import jax
import jax.numpy as jnp
import numpy as np
from jax import lax
from jax.experimental import pallas as pl
from jax.experimental.pallas import tpu as pltpu
from jax.experimental.pallas import tpu_sc as plsc

M = 512
K = 32
NUM_FREQ = 10
N = 20000
NPAD = 20480
ROWS = 160

_INTERPRET = False


def _fps_body(init_ref, pxs_ref, pys_ref, px_ref, py_ref, cent_ref, dist_ref):
    px = px_ref[:]
    py = py_ref[:]
    row = lax.broadcasted_iota(jnp.int32, (ROWS, 128), 0)
    col = lax.broadcasted_iota(jnp.int32, (ROWS, 128), 1)
    fi = row * 128 + col
    valid = fi < N

    init = init_ref[0]
    cx0 = pxs_ref[init]
    cy0 = pys_ref[init]
    cent_ref[0, 0] = cx0
    cent_ref[0, 1] = cy0
    dist_ref[:] = jnp.where(valid, jnp.inf, 0.0)

    def body(i, carry):
        cx, cy = carry
        dx = px - cx
        dy = py - cy
        nd2 = dx * dx + dy * dy + jnp.float32(1e-12)
        d2 = jnp.minimum(dist_ref[:], nd2)
        s = jnp.sqrt(d2)
        mx = jnp.max(s)
        idx = jnp.min(jnp.where(s == mx, fi, NPAD))
        m2 = fi == idx
        dist_ref[:] = jnp.where(m2, 0.0, d2)
        ncx = pxs_ref[idx]
        ncy = pys_ref[idx]
        cent_ref[i, 0] = ncx
        cent_ref[i, 1] = ncy
        return (ncx, ncy)

    lax.fori_loop(1, M, body, (cx0, cy0))


def _fps(px, py, pxflat, pyflat, init):
    return pl.pallas_call(
        _fps_body,
        out_shape=jax.ShapeDtypeStruct((M, 2), jnp.float32),
        in_specs=[
            pl.BlockSpec(memory_space=pltpu.SMEM),
            pl.BlockSpec(memory_space=pltpu.SMEM),
            pl.BlockSpec(memory_space=pltpu.SMEM),
            pl.BlockSpec(memory_space=pltpu.VMEM),
            pl.BlockSpec(memory_space=pltpu.VMEM),
        ],
        out_specs=pl.BlockSpec(memory_space=pltpu.SMEM),
        scratch_shapes=[pltpu.VMEM((ROWS, 128), jnp.float32)],
        interpret=_INTERPRET,
    )(init, pxflat, pyflat, px, py)


_NB = 1024


def _nn_body(px_ref, py_ref, cx_ref, cy_ref, out_ref):
    dx = px_ref[:] - cx_ref[:]
    dy = py_ref[:] - cy_ref[:]
    d2 = dx * dx + dy * dy
    m = jnp.min(d2, axis=1, keepdims=True)
    lane = lax.broadcasted_iota(jnp.int32, (_NB, M), 1)
    out_ref[:] = jnp.min(jnp.where(d2 == m, lane, M), axis=1, keepdims=True)


def _nearest(px_col, py_col, cx_row, cy_row):
    grid = NPAD // _NB
    return pl.pallas_call(
        _nn_body,
        grid=(grid,),
        out_shape=jax.ShapeDtypeStruct((NPAD, 1), jnp.int32),
        in_specs=[
            pl.BlockSpec((_NB, 1), lambda i: (i, 0)),
            pl.BlockSpec((_NB, 1), lambda i: (i, 0)),
            pl.BlockSpec((1, M), lambda i: (0, 0)),
            pl.BlockSpec((1, M), lambda i: (0, 0)),
        ],
        out_specs=pl.BlockSpec((_NB, 1), lambda i: (i, 0)),
        interpret=_INTERPRET,
    )(px_col, py_col, cx_row, cy_row)


_NW = 16
_CH = NPAD // _NW
_CPW = M // _NW
_SELPAD = M * K + 256


def _sc_group_body(near_hbm, ptx_hbm, pty_hbm, cx_hbm, cy_hbm,
                   relx_hbm, rely_hbm,
                   near_v, hist_v, allhist_v, start_v, cnts_v,
                   sel_local, merge_v, selblk_v, ptx_v, pty_v,
                   cxl_v, cyl_v, relx_v, rely_v,
                   hist_sh, sel_sh, sem1, sem2):
    wid = lax.axis_index("s")
    base = wid * _CH
    cp1 = pltpu.async_copy(ptx_hbm, ptx_v, sem1)
    cp2 = pltpu.async_copy(pty_hbm, pty_v, sem2)
    pltpu.sync_copy(near_hbm.at[pl.ds(base, _CH)], near_v)
    pltpu.sync_copy(cx_hbm.at[pl.ds(wid * _CPW, _CPW)], cxl_v)
    pltpu.sync_copy(cy_hbm.at[pl.ds(wid * _CPW, _CPW)], cyl_v)

    zero16 = jnp.zeros((16,), jnp.int32)
    lane = lax.broadcasted_iota(jnp.int32, (16,), 0)
    for j in range(M // 16):
        hist_v[pl.ds(j * 16, 16)] = zero16

    def habody(t, carry):
        c16 = near_v[pl.ds(t * 16, 16)]
        validm = c16 < M
        cm16 = jnp.minimum(c16, M - 1)
        cnt16, lastm = plsc.scan_count(cm16, validm)
        h16 = plsc.load_gather(hist_v, [cm16])
        plsc.store_scatter(hist_v, [cm16], h16 + cnt16, mask=lastm)
        return carry

    lax.fori_loop(0, _CH // 16, habody, 0)

    pltpu.sync_copy(hist_v, hist_sh.at[wid])
    plsc.subcore_barrier()
    pltpu.sync_copy(hist_sh, allhist_v)

    for j in range(M // 16):
        start_v[pl.ds(j * 16, 16)] = zero16
        cnts_v[pl.ds(j * 16, 16)] = zero16

    def pbody(t, carry):
        mlt = (t < wid).astype(jnp.int32)
        for j in range(M // 16):
            sl = pl.ds(j * 16, 16)
            v = allhist_v[t, sl]
            cnts_v[sl] = cnts_v[sl] + v
            start_v[sl] = start_v[sl] + v * mlt
        return carry

    lax.fori_loop(0, _NW, pbody, 0)

    def zbody(t, carry):
        for h in range(8):
            sel_local[pl.ds(t * 128 + h * 16, 16)] = zero16
        return carry

    lax.fori_loop(0, M * K // 128, zbody, 0)

    def bbody(t, carry):
        c16 = near_v[pl.ds(t * 16, 16)]
        validm = c16 < M
        cm16 = jnp.minimum(c16, M - 1)
        cnt16, lastm = plsc.scan_count(cm16, validm)
        s16 = plsc.load_gather(start_v, [cm16])
        r16 = s16 + cnt16 - 1
        plsc.store_scatter(start_v, [cm16], s16 + cnt16, mask=lastm)
        take = jnp.logical_and(validm, r16 < K)
        slot16 = cm16 * K + jnp.minimum(r16, K - 1)
        val16 = base + t * 16 + lane
        plsc.store_scatter(sel_local, [slot16], val16, mask=take)
        return carry

    lax.fori_loop(0, _CH // 16, bbody, 0)

    pltpu.sync_copy(sel_local, sel_sh.at[wid])
    plsc.subcore_barrier()

    pltpu.sync_copy(sel_sh.at[:, pl.ds(wid * _CPW * K, _CPW * K)], merge_v)

    def mbody(j, carry):
        acc = merge_v[0, pl.ds(j * 16, 16)]
        for t in range(1, _NW):
            acc = acc + merge_v[t, pl.ds(j * 16, 16)]
        selblk_v[pl.ds(j * 16, 16)] = acc
        return carry

    lax.fori_loop(0, _CPW * K // 16, mbody, 0)
    cp1.wait()
    cp2.wait()

    cxl0 = cxl_v[pl.ds(0, 16)]
    cxl1 = cxl_v[pl.ds(16, 16)]
    cyl0 = cyl_v[pl.ds(0, 16)]
    cyl1 = cyl_v[pl.ds(16, 16)]
    for cl in range(_CPW):
        cnt = plsc.load_gather(cnts_v, [zero16 + wid * _CPW + cl])[0]
        cntc = jnp.minimum(cnt, K)
        last = plsc.load_gather(
            selblk_v, [zero16 + cl * K + jnp.maximum(cntc - 1, 0)])[0]
        cx = cxl0[cl] if cl < 16 else cxl1[cl - 16]
        cy = cyl0[cl] if cl < 16 else cyl1[cl - 16]
        emptyc = cnt == 0
        for j in range(K // 16):
            off = pl.ds(cl * K + j * 16, 16)
            kvec = lane + j * 16
            s = jnp.where(kvec < cntc, selblk_v[off], last)
            s = jnp.clip(s, 0, N - 1)
            gx = plsc.load_gather(ptx_v, [s])
            gy = plsc.load_gather(pty_v, [s])
            first = jnp.logical_and(emptyc, kvec == 0)
            rx = jnp.where(first, 0.0, jnp.where(emptyc, 0.0 - cx, gx - cx))
            ry = jnp.where(first, 0.0, jnp.where(emptyc, 0.0 - cy, gy - cy))
            relx_v[off] = rx
            rely_v[off] = ry

    pltpu.sync_copy(relx_v, relx_hbm.at[pl.ds(wid * _CPW * K, _CPW * K)])
    pltpu.sync_copy(rely_v, rely_hbm.at[pl.ds(wid * _CPW * K, _CPW * K)])


def _sc_group(near_pad, ptx_pad, pty_pad, centx, centy):
    mesh = plsc.VectorSubcoreMesh(core_axis_name="c", subcore_axis_name="s",
                                  num_cores=1)
    return pl.kernel(
        _sc_group_body,
        out_type=[jax.ShapeDtypeStruct((M * K,), jnp.float32),
                  jax.ShapeDtypeStruct((M * K,), jnp.float32)],
        mesh=mesh,
        compiler_params=pltpu.CompilerParams(needs_layout_passes=False),
        scratch_types=[
            pltpu.VMEM((_CH,), jnp.int32),
            pltpu.VMEM((M,), jnp.int32),
            pltpu.VMEM((_NW, M), jnp.int32),
            pltpu.VMEM((M,), jnp.int32),
            pltpu.VMEM((M,), jnp.int32),
            pltpu.VMEM((M * K,), jnp.int32),
            pltpu.VMEM((_NW, _CPW * K), jnp.int32),
            pltpu.VMEM((_CPW * K,), jnp.int32),
            pltpu.VMEM((NPAD,), jnp.float32),
            pltpu.VMEM((NPAD,), jnp.float32),
            pltpu.VMEM((_CPW,), jnp.float32),
            pltpu.VMEM((_CPW,), jnp.float32),
            pltpu.VMEM((_CPW * K,), jnp.float32),
            pltpu.VMEM((_CPW * K,), jnp.float32),
            pltpu.VMEM_SHARED((_NW, M), jnp.int32),
            pltpu.VMEM_SHARED((_NW, M * K), jnp.int32),
            pltpu.SemaphoreType.DMA,
            pltpu.SemaphoreType.DMA,
        ],
    )(near_pad, ptx_pad, pty_pad, centx, centy)


def _mlp_body(relx_ref, rely_ref, w1_ref, b1_ref, g1_ref, be1_ref,
              w2_ref, b2_ref, g2_ref, be2_ref, w3_ref, b3_ref, out_ref):
    rx = relx_ref[:]
    ry = rely_ref[:]
    ii = lax.broadcasted_iota(jnp.int32, (1, 2 * NUM_FREQ), 1)
    freq = lax.shift_left(1, ii // 2).astype(jnp.float32) * jnp.float32(np.pi)
    rsel = jnp.where((ii % 2) == 0, rx, ry)
    args = rsel * freq
    x = jnp.concatenate([jnp.sin(args), jnp.cos(args)], axis=1)

    def dense_bn_relu(h, w_ref, b_ref, g_ref, be_ref):
        h = jnp.dot(h, w_ref[:], preferred_element_type=jnp.float32) + b_ref[:]
        mu = jnp.mean(h, axis=0, keepdims=True)
        c = h - mu
        v = jnp.mean(c * c, axis=0, keepdims=True)
        h = g_ref[:] * c / jnp.sqrt(v + 1e-5) + be_ref[:]
        return jnp.maximum(h, 0.0)

    h = dense_bn_relu(x, w1_ref, b1_ref, g1_ref, be1_ref)
    h = dense_bn_relu(h, w2_ref, b2_ref, g2_ref, be2_ref)
    h = jnp.dot(h, w3_ref[:], preferred_element_type=jnp.float32) + b3_ref[:]
    out_ref[:] = jnp.max(h.reshape(M, K, 16), axis=1)


def _mlp(relx, rely, w1p, b1, g1, be1, w2, b2, g2, be2, w3, b3):
    args = (relx, rely, w1p, b1.reshape(1, -1), g1.reshape(1, -1),
            be1.reshape(1, -1), w2, b2.reshape(1, -1), g2.reshape(1, -1),
            be2.reshape(1, -1), w3, b3.reshape(1, -1))
    return pl.pallas_call(
        _mlp_body,
        out_shape=jax.ShapeDtypeStruct((M, 16), jnp.float32),
        interpret=_INTERPRET,
    )(*args)


def kernel(points, W1, b1, g1, be1, W2, b2, g2, be2, W3, b3):
    key = jax.random.key(42)
    init_idx = jax.random.randint(key, (1,), 0, N).astype(jnp.int32)

    ptx = points[:, 0]
    pty = points[:, 1]
    px2 = jnp.pad(ptx, (0, NPAD - N)).reshape(ROWS, 128)
    py2 = jnp.pad(pty, (0, NPAD - N)).reshape(ROWS, 128)
    cent = _fps(px2, py2, px2.reshape(NPAD), py2.reshape(NPAD),
                init_idx)

    px_col = px2.reshape(NPAD, 1)
    py_col = py2.reshape(NPAD, 1)
    cx_row = cent[:, 0].reshape(1, M)
    cy_row = cent[:, 1].reshape(1, M)
    nearest = _nearest(px_col, py_col, cx_row, cy_row)[:, 0]

    near_pad = jnp.where(jnp.arange(NPAD, dtype=jnp.int32) < N, nearest, M)
    relx_f, rely_f = _sc_group(near_pad, px2.reshape(NPAD), py2.reshape(NPAD),
                               cent[:, 0], cent[:, 1])
    relx = relx_f.reshape(M * K, 1)
    rely = rely_f.reshape(M * K, 1)

    perm = ([4 * fe + 2 * d for fe in range(NUM_FREQ) for d in range(2)]
            + [4 * fe + 2 * d + 1 for fe in range(NUM_FREQ) for d in range(2)])
    w1p = W1[jnp.asarray(perm), :]

    feat = _mlp(relx, rely, w1p, b1, g1, be1, W2, b2, g2, be2, W3, b3)
    return feat, cent


# speedup vs baseline: 1.6876x; 1.0021x over previous
import jax
import jax.numpy as jnp
import numpy as np
from jax import lax
from jax.experimental import pallas as pl
from jax.experimental.pallas import tpu as pltpu
from jax.experimental.pallas import tpu_sc as plsc

M = 512
K = 32
NUM_FREQ = 10
N = 20000
NPAD = 20480
ROWS = 160


def _fps_body(init_ref, pxs_ref, pys_ref, px_ref, py_ref, cent_ref, dist_ref):
    px = px_ref[:]
    py = py_ref[:]
    row = lax.broadcasted_iota(jnp.int32, (ROWS, 128), 0)
    col = lax.broadcasted_iota(jnp.int32, (ROWS, 128), 1)
    fi = row * 128 + col
    valid = fi < N

    init = init_ref[0]
    cx0 = pxs_ref[init]
    cy0 = pys_ref[init]
    cent_ref[0, 0] = cx0
    cent_ref[0, 1] = cy0
    dist_ref[:] = jnp.where(valid, jnp.inf, 0.0)

    def body(i, carry):
        cx, cy = carry
        dx = px - cx
        dy = py - cy
        nd2 = dx * dx + dy * dy + jnp.float32(1e-12)
        d2 = jnp.minimum(dist_ref[:], nd2)
        s = jnp.sqrt(d2)
        mx = jnp.max(s)
        idx = jnp.min(jnp.where(s == mx, fi, NPAD))
        m2 = fi == idx
        dist_ref[:] = jnp.where(m2, 0.0, d2)
        ncx = pxs_ref[idx]
        ncy = pys_ref[idx]
        cent_ref[i, 0] = ncx
        cent_ref[i, 1] = ncy
        return (ncx, ncy)

    lax.fori_loop(1, M, body, (cx0, cy0))


def _fps(px, py, pxflat, pyflat, init):
    return pl.pallas_call(
        _fps_body,
        out_shape=jax.ShapeDtypeStruct((M, 2), jnp.float32),
        in_specs=[
            pl.BlockSpec(memory_space=pltpu.SMEM),
            pl.BlockSpec(memory_space=pltpu.SMEM),
            pl.BlockSpec(memory_space=pltpu.SMEM),
            pl.BlockSpec(memory_space=pltpu.VMEM),
            pl.BlockSpec(memory_space=pltpu.VMEM),
        ],
        out_specs=pl.BlockSpec(memory_space=pltpu.SMEM),
        scratch_shapes=[pltpu.VMEM((ROWS, 128), jnp.float32)],

    )(init, pxflat, pyflat, px, py)


_NB = 1024


def _nn_body(px_ref, py_ref, cx_ref, cy_ref, out_ref):
    dx = px_ref[:] - cx_ref[:]
    dy = py_ref[:] - cy_ref[:]
    d2 = dx * dx + dy * dy
    m = jnp.min(d2, axis=1, keepdims=True)
    lane = lax.broadcasted_iota(jnp.int32, (_NB, M), 1)
    out_ref[:] = jnp.min(jnp.where(d2 == m, lane, M), axis=1, keepdims=True)


def _nearest(px_col, py_col, cx_row, cy_row):
    grid = NPAD // _NB
    return pl.pallas_call(
        _nn_body,
        grid=(grid,),
        out_shape=jax.ShapeDtypeStruct((NPAD, 1), jnp.int32),
        in_specs=[
            pl.BlockSpec((_NB, 1), lambda i: (i, 0)),
            pl.BlockSpec((_NB, 1), lambda i: (i, 0)),
            pl.BlockSpec((1, M), lambda i: (0, 0)),
            pl.BlockSpec((1, M), lambda i: (0, 0)),
        ],
        out_specs=pl.BlockSpec((_NB, 1), lambda i: (i, 0)),

    )(px_col, py_col, cx_row, cy_row)


_NW = 16
_CH = NPAD // _NW
_CPW = M // _NW


def _sc_group_body(near_hbm, ptx_hbm, pty_hbm, cx_hbm, cy_hbm,
                   relx_hbm, rely_hbm,
                   near_v, hist_v, allhist_v, start_v, cnts_v,
                   sel_local, merge_v, selblk_v, ptx_v, pty_v,
                   cxl_v, cyl_v, relx_v, rely_v,
                   hist_sh, sel_sh, sem1, sem2):
    wid = lax.axis_index("s")
    base = wid * _CH
    cp1 = pltpu.async_copy(ptx_hbm, ptx_v, sem1)
    cp2 = pltpu.async_copy(pty_hbm, pty_v, sem2)
    pltpu.sync_copy(near_hbm.at[pl.ds(base, _CH)], near_v)
    pltpu.sync_copy(cx_hbm.at[pl.ds(wid * _CPW, _CPW)], cxl_v)
    pltpu.sync_copy(cy_hbm.at[pl.ds(wid * _CPW, _CPW)], cyl_v)

    zero16 = jnp.zeros((16,), jnp.int32)
    lane = lax.broadcasted_iota(jnp.int32, (16,), 0)
    for j in range(M // 16):
        hist_v[pl.ds(j * 16, 16)] = zero16

    def habody(t, carry):
        c16 = near_v[pl.ds(t * 16, 16)]
        validm = c16 < M
        cm16 = jnp.minimum(c16, M - 1)
        cnt16, lastm = plsc.scan_count(cm16, validm)
        h16 = plsc.load_gather(hist_v, [cm16])
        plsc.store_scatter(hist_v, [cm16], h16 + cnt16, mask=lastm)
        return carry

    lax.fori_loop(0, _CH // 16, habody, 0)

    pltpu.sync_copy(hist_v, hist_sh.at[wid])
    plsc.subcore_barrier()
    pltpu.sync_copy(hist_sh, allhist_v)

    for j in range(M // 16):
        start_v[pl.ds(j * 16, 16)] = zero16
        cnts_v[pl.ds(j * 16, 16)] = zero16

    def pbody(t, carry):
        mlt = (t < wid).astype(jnp.int32)
        for j in range(M // 16):
            sl = pl.ds(j * 16, 16)
            v = allhist_v[t, sl]
            cnts_v[sl] = cnts_v[sl] + v
            start_v[sl] = start_v[sl] + v * mlt
        return carry

    lax.fori_loop(0, _NW, pbody, 0)

    def zbody(t, carry):
        for h in range(8):
            sel_local[pl.ds(t * 128 + h * 16, 16)] = zero16
        return carry

    lax.fori_loop(0, M * K // 128, zbody, 0)

    def bbody(t, carry):
        c16 = near_v[pl.ds(t * 16, 16)]
        validm = c16 < M
        cm16 = jnp.minimum(c16, M - 1)
        cnt16, lastm = plsc.scan_count(cm16, validm)
        s16 = plsc.load_gather(start_v, [cm16])
        r16 = s16 + cnt16 - 1
        plsc.store_scatter(start_v, [cm16], s16 + cnt16, mask=lastm)
        take = jnp.logical_and(validm, r16 < K)
        slot16 = cm16 * K + jnp.minimum(r16, K - 1)
        val16 = base + t * 16 + lane
        plsc.store_scatter(sel_local, [slot16], val16, mask=take)
        return carry

    lax.fori_loop(0, _CH // 16, bbody, 0)

    pltpu.sync_copy(sel_local, sel_sh.at[wid])
    plsc.subcore_barrier()

    pltpu.sync_copy(sel_sh.at[:, pl.ds(wid * _CPW * K, _CPW * K)], merge_v)

    def mbody(j, carry):
        acc = merge_v[0, pl.ds(j * 16, 16)]
        for t in range(1, _NW):
            acc = acc + merge_v[t, pl.ds(j * 16, 16)]
        selblk_v[pl.ds(j * 16, 16)] = acc
        return carry

    lax.fori_loop(0, _CPW * K // 16, mbody, 0)
    cp1.wait()
    cp2.wait()

    cxl0 = cxl_v[pl.ds(0, 16)]
    cxl1 = cxl_v[pl.ds(16, 16)]
    cyl0 = cyl_v[pl.ds(0, 16)]
    cyl1 = cyl_v[pl.ds(16, 16)]
    for cl in range(_CPW):
        cnt = plsc.load_gather(cnts_v, [zero16 + wid * _CPW + cl])[0]
        cntc = jnp.minimum(cnt, K)
        last = plsc.load_gather(
            selblk_v, [zero16 + cl * K + jnp.maximum(cntc - 1, 0)])[0]
        cx = cxl0[cl] if cl < 16 else cxl1[cl - 16]
        cy = cyl0[cl] if cl < 16 else cyl1[cl - 16]
        emptyc = cnt == 0
        for j in range(K // 16):
            off = pl.ds(cl * K + j * 16, 16)
            kvec = lane + j * 16
            s = jnp.where(kvec < cntc, selblk_v[off], last)
            s = jnp.clip(s, 0, N - 1)
            gx = plsc.load_gather(ptx_v, [s])
            gy = plsc.load_gather(pty_v, [s])
            first = jnp.logical_and(emptyc, kvec == 0)
            rx = jnp.where(first, 0.0, jnp.where(emptyc, 0.0 - cx, gx - cx))
            ry = jnp.where(first, 0.0, jnp.where(emptyc, 0.0 - cy, gy - cy))
            relx_v[off] = rx
            rely_v[off] = ry

    pltpu.sync_copy(relx_v, relx_hbm.at[pl.ds(wid * _CPW * K, _CPW * K)])
    pltpu.sync_copy(rely_v, rely_hbm.at[pl.ds(wid * _CPW * K, _CPW * K)])


def _sc_group(near_pad, ptx_pad, pty_pad, centx, centy):
    mesh = plsc.VectorSubcoreMesh(core_axis_name="c", subcore_axis_name="s",
                                  num_cores=1)
    return pl.kernel(
        _sc_group_body,
        out_type=[jax.ShapeDtypeStruct((M * K,), jnp.float32),
                  jax.ShapeDtypeStruct((M * K,), jnp.float32)],
        mesh=mesh,
        compiler_params=pltpu.CompilerParams(needs_layout_passes=False),
        scratch_types=[
            pltpu.VMEM((_CH,), jnp.int32),
            pltpu.VMEM((M,), jnp.int32),
            pltpu.VMEM((_NW, M), jnp.int32),
            pltpu.VMEM((M,), jnp.int32),
            pltpu.VMEM((M,), jnp.int32),
            pltpu.VMEM((M * K,), jnp.int32),
            pltpu.VMEM((_NW, _CPW * K), jnp.int32),
            pltpu.VMEM((_CPW * K,), jnp.int32),
            pltpu.VMEM((NPAD,), jnp.float32),
            pltpu.VMEM((NPAD,), jnp.float32),
            pltpu.VMEM((_CPW,), jnp.float32),
            pltpu.VMEM((_CPW,), jnp.float32),
            pltpu.VMEM((_CPW * K,), jnp.float32),
            pltpu.VMEM((_CPW * K,), jnp.float32),
            pltpu.VMEM_SHARED((_NW, M), jnp.int32),
            pltpu.VMEM_SHARED((_NW, M * K), jnp.int32),
            pltpu.SemaphoreType.DMA,
            pltpu.SemaphoreType.DMA,
        ],
    )(near_pad, ptx_pad, pty_pad, centx, centy)


def _mlp_body(relx_ref, rely_ref, w1_ref, b1_ref, g1_ref, be1_ref,
              w2_ref, b2_ref, g2_ref, be2_ref, w3_ref, b3_ref, out_ref):
    rx = relx_ref[:]
    ry = rely_ref[:]
    ii = lax.broadcasted_iota(jnp.int32, (1, 2 * NUM_FREQ), 1)
    freq = lax.shift_left(1, ii // 2).astype(jnp.float32) * jnp.float32(np.pi)
    rsel = jnp.where((ii % 2) == 0, rx, ry)
    args = rsel * freq
    x = jnp.concatenate([jnp.sin(args), jnp.cos(args)], axis=1)

    def dense_bn_relu(h, w_ref, b_ref, g_ref, be_ref):
        h = jnp.dot(h, w_ref[:], preferred_element_type=jnp.float32) + b_ref[:]
        mu = jnp.mean(h, axis=0, keepdims=True)
        c = h - mu
        v = jnp.mean(c * c, axis=0, keepdims=True)
        h = g_ref[:] * c / jnp.sqrt(v + 1e-5) + be_ref[:]
        return jnp.maximum(h, 0.0)

    h = dense_bn_relu(x, w1_ref, b1_ref, g1_ref, be1_ref)
    h = dense_bn_relu(h, w2_ref, b2_ref, g2_ref, be2_ref)
    h = jnp.dot(h, w3_ref[:], preferred_element_type=jnp.float32) + b3_ref[:]
    out_ref[:] = jnp.max(h.reshape(M, K, 16), axis=1)


def _mlp(relx, rely, w1p, b1, g1, be1, w2, b2, g2, be2, w3, b3):
    args = (relx, rely, w1p, b1.reshape(1, -1), g1.reshape(1, -1),
            be1.reshape(1, -1), w2, b2.reshape(1, -1), g2.reshape(1, -1),
            be2.reshape(1, -1), w3, b3.reshape(1, -1))
    return pl.pallas_call(
        _mlp_body,
        out_shape=jax.ShapeDtypeStruct((M, 16), jnp.float32),

    )(*args)


def kernel(points, W1, b1, g1, be1, W2, b2, g2, be2, W3, b3):
    key = jax.random.key(42)
    init_idx = jax.random.randint(key, (1,), 0, N).astype(jnp.int32)

    ptx = points[:, 0]
    pty = points[:, 1]
    px2 = jnp.pad(ptx, (0, NPAD - N)).reshape(ROWS, 128)
    py2 = jnp.pad(pty, (0, NPAD - N)).reshape(ROWS, 128)
    cent = _fps(px2, py2, px2.reshape(NPAD), py2.reshape(NPAD),
                init_idx)

    px_col = px2.reshape(NPAD, 1)
    py_col = py2.reshape(NPAD, 1)
    cx_row = cent[:, 0].reshape(1, M)
    cy_row = cent[:, 1].reshape(1, M)
    nearest = _nearest(px_col, py_col, cx_row, cy_row)[:, 0]

    near_pad = jnp.where(jnp.arange(NPAD, dtype=jnp.int32) < N, nearest, M)
    relx_f, rely_f = _sc_group(near_pad, px2.reshape(NPAD), py2.reshape(NPAD),
                               cent[:, 0], cent[:, 1])
    relx = relx_f.reshape(M * K, 1)
    rely = rely_f.reshape(M * K, 1)

    perm = ([4 * fe + 2 * d for fe in range(NUM_FREQ) for d in range(2)]
            + [4 * fe + 2 * d + 1 for fe in range(NUM_FREQ) for d in range(2)])
    w1p = W1[jnp.asarray(perm), :]

    feat = _mlp(relx, rely, w1p, b1, g1, be1, W2, b2, g2, be2, W3, b3)
    return feat, cent
